# Initial kernel scaffold; baseline (speedup 1.0000x reference)
import jax
import jax.numpy as jnp
from jax.experimental import pallas as pl


def kernel(stu_tensor, tea_tensor, stu_uncertainty):
    raise NotImplementedError("write your pallas kernel here")


# speedup vs baseline: 12.5578x; 12.5578x over previous
import dataclasses
import functools

import jax
import jax.numpy as jnp
from jax import lax
from jax.experimental import pallas as pl
from jax.experimental.pallas import tpu as pltpu
from jax.experimental.pallas import tpu_sc as plsc

NB = 32
EPS = 1e-08
H = W = 512
N = H * W
NWORK = 32
SHARD = N // NWORK
LANES = 16
HIST = NB * NB


def _sc_hist_body(stu_hbm, tea_hbm, unc_hbm, out_hbm,
                  s_v, t_v, u_v, hist_v, red_v, sem):
    wid = lax.axis_index("s") * 2 + lax.axis_index("c")
    base = wid * SHARD

    zeros = jnp.zeros((LANES,), jnp.float32)

    @pl.loop(0, LANES * HIST, step=LANES)
    def _(i):
        hist_v[pl.ds(i, LANES)] = zeros

    c1 = pltpu.async_copy(stu_hbm.at[pl.ds(base, SHARD)], s_v, sem)
    c2 = pltpu.async_copy(tea_hbm.at[pl.ds(base, SHARD)], t_v, sem)
    c3 = pltpu.async_copy(unc_hbm.at[pl.ds(base, SHARD)], u_v, sem)
    c1.wait()
    c2.wait()
    c3.wait()

    lane_off = lax.iota(jnp.int32, LANES) * HIST
    ones = jnp.ones((LANES,), jnp.float32)

    @pl.loop(0, SHARD, step=LANES)
    def _(j):
        s = s_v[pl.ds(j, LANES)]
        t = t_v[pl.ds(j, LANES)]
        u = u_v[pl.ds(j, LANES)]
        topo = 1.0 - jnp.abs(s - t)
        conf = 1.0 - u
        ti = jnp.minimum((topo * float(NB)).astype(jnp.int32), NB - 1)
        cj = jnp.minimum((conf * float(NB)).astype(jnp.int32), NB - 1)
        valid = (topo < 1.0) & (conf < 1.0)
        idx = ti * NB + cj + lane_off
        plsc.addupdate_scatter(hist_v, [idx], ones, mask=valid)

    @pl.loop(0, HIST, step=LANES)
    def _(b):
        acc = hist_v[pl.ds(b, LANES)]
        for l in range(1, LANES):
            acc = acc + hist_v[pl.ds(l * HIST + b, LANES)]
        red_v[pl.ds(b, LANES)] = acc

    pltpu.sync_copy(red_v, out_hbm.at[wid])


def _sc_hist(stu_f, tea_f, unc_f):
    mesh = plsc.VectorSubcoreMesh(core_axis_name="c", subcore_axis_name="s")
    cp = pltpu.CompilerParams()
    if "needs_layout_passes" in pltpu.CompilerParams.__dataclass_fields__:
        cp = dataclasses.replace(cp, needs_layout_passes=False)
    return pl.kernel(
        _sc_hist_body,
        out_type=jax.ShapeDtypeStruct((NWORK, HIST), jnp.float32),
        mesh=mesh,
        scratch_types=[
            pltpu.VMEM((SHARD,), jnp.float32),
            pltpu.VMEM((SHARD,), jnp.float32),
            pltpu.VMEM((SHARD,), jnp.float32),
            pltpu.VMEM((LANES * HIST,), jnp.float32),
            pltpu.VMEM((HIST,), jnp.float32),
            pltpu.SemaphoreType.DMA,
        ],
        compiler_params=cp,
    )(stu_f, tea_f, unc_f)


def _tc_ent_kernel(stu_ref, tea_ref, unc_ref, sum_ref, cnt_ref):
    topo = 1.0 - jnp.abs(stu_ref[...] - tea_ref[...])
    conf = 1.0 - unc_ref[...]
    mask = (conf > 0.8).astype(jnp.float32)
    tc = jnp.clip(topo, EPS, 1.0 - EPS)
    ent = -(tc * jnp.log(tc + EPS))
    sum_ref[0, 0] = jnp.sum(ent * mask)
    cnt_ref[0, 0] = jnp.sum(mask)


def _tc_ent(stu, tea, unc):
    return pl.pallas_call(
        _tc_ent_kernel,
        out_shape=(jax.ShapeDtypeStruct((1, 1), jnp.float32),
                   jax.ShapeDtypeStruct((1, 1), jnp.float32)),
        out_specs=(pl.BlockSpec(memory_space=pltpu.SMEM),
                   pl.BlockSpec(memory_space=pltpu.SMEM)),
    )(stu, tea, unc)


def _tc_final_kernel(parts_ref, esum_ref, ecnt_ref, out_ref):
    hist = jnp.sum(parts_ref[...], axis=0)
    total = jnp.sum(hist)
    jp = hist / (total + EPS)
    mt = jnp.sum(jp, axis=1, keepdims=True)
    mc = jnp.sum(jp, axis=0, keepdims=True)
    outer = mt * mc
    contrib = jnp.where(jp > EPS, jp * jnp.log(jp / (outer + EPS) + EPS), 0.0)
    mi = jnp.sum(contrib)
    mi = jnp.where(total < EPS, jnp.float32(0.0), mi)
    esum = esum_ref[0, 0]
    cnt = ecnt_ref[0, 0]
    ent_loss = jnp.where(cnt > 10.0, esum / jnp.maximum(cnt, 1.0),
                         jnp.float32(0.0))
    out_ref[0, 0] = -mi + 0.1 * ent_loss


def _tc_final(parts, esum, ecnt):
    return pl.pallas_call(
        _tc_final_kernel,
        out_shape=jax.ShapeDtypeStruct((1, 1), jnp.float32),
        in_specs=(pl.BlockSpec(),
                  pl.BlockSpec(memory_space=pltpu.SMEM),
                  pl.BlockSpec(memory_space=pltpu.SMEM)),
        out_specs=pl.BlockSpec(memory_space=pltpu.SMEM),
    )(parts, esum, ecnt)


def kernel(stu_tensor, tea_tensor, stu_uncertainty):
    stu_f = stu_tensor.reshape(-1)
    tea_f = tea_tensor.reshape(-1)
    unc_f = stu_uncertainty.reshape(-1)
    parts = _sc_hist(stu_f, tea_f, unc_f)
    esum, ecnt = _tc_ent(stu_tensor, tea_tensor, stu_uncertainty)
    parts3 = parts.reshape(NWORK, NB, NB)
    out = _tc_final(parts3, esum, ecnt)
    return out.reshape(())


# speedup vs baseline: 14.3726x; 1.1445x over previous
import dataclasses
import functools

import jax
import jax.numpy as jnp
from jax import lax
from jax.experimental import pallas as pl
from jax.experimental.pallas import tpu as pltpu
from jax.experimental.pallas import tpu_sc as plsc

NB = 32
EPS = 1e-08
H = W = 512
N = H * W
NWORK = 32
SHARD = N // NWORK
LANES = 16
HIST = NB * NB


def _sc_hist_body(stu_hbm, tea_hbm, unc_hbm, out_hbm,
                  s_v, t_v, u_v, hist_v, red_v, sem):
    wid = lax.axis_index("s") * 2 + lax.axis_index("c")
    base = wid * SHARD

    c1 = pltpu.async_copy(stu_hbm.at[pl.ds(base, SHARD)], s_v, sem)
    c2 = pltpu.async_copy(tea_hbm.at[pl.ds(base, SHARD)], t_v, sem)
    c3 = pltpu.async_copy(unc_hbm.at[pl.ds(base, SHARD)], u_v, sem)

    zeros = jnp.zeros((LANES,), jnp.float32)

    @pl.loop(0, LANES * HIST, step=LANES * 16)
    def _(i):
        for k in range(16):
            hist_v[pl.ds(i + k * LANES, LANES)] = zeros

    c1.wait()
    c2.wait()
    c3.wait()

    lane_off = lax.iota(jnp.int32, LANES) * HIST
    ones = jnp.ones((LANES,), jnp.float32)
    nbf = float(NB)

    UNROLL = 8

    @pl.loop(0, SHARD, step=LANES * UNROLL)
    def _(j):
        for k in range(UNROLL):
            o = j + k * LANES
            s = s_v[pl.ds(o, LANES)]
            t = t_v[pl.ds(o, LANES)]
            u = u_v[pl.ds(o, LANES)]
            a = jnp.abs(s - t)
            ti = (nbf - nbf * a).astype(jnp.int32)
            cj = (nbf - nbf * u).astype(jnp.int32)
            valid = (a > 0.0) & (u > 0.0)
            idx = ti * NB + cj + lane_off
            plsc.addupdate_scatter(hist_v, [idx], ones, mask=valid)

    @pl.loop(0, HIST, step=LANES)
    def _(b):
        vs = [hist_v[pl.ds(l * HIST + b, LANES)] for l in range(LANES)]
        while len(vs) > 1:
            nxt = [vs[i] + vs[i + 1] for i in range(0, len(vs) - 1, 2)]
            if len(vs) % 2:
                nxt.append(vs[-1])
            vs = nxt
        red_v[pl.ds(b, LANES)] = vs[0]

    pltpu.sync_copy(red_v, out_hbm.at[wid])


def _sc_hist(stu_f, tea_f, unc_f):
    mesh = plsc.VectorSubcoreMesh(core_axis_name="c", subcore_axis_name="s")
    cp = pltpu.CompilerParams()
    if "needs_layout_passes" in pltpu.CompilerParams.__dataclass_fields__:
        cp = dataclasses.replace(cp, needs_layout_passes=False)
    return pl.kernel(
        _sc_hist_body,
        out_type=jax.ShapeDtypeStruct((NWORK, HIST), jnp.float32),
        mesh=mesh,
        scratch_types=[
            pltpu.VMEM((SHARD,), jnp.float32),
            pltpu.VMEM((SHARD,), jnp.float32),
            pltpu.VMEM((SHARD,), jnp.float32),
            pltpu.VMEM((LANES * HIST,), jnp.float32),
            pltpu.VMEM((HIST,), jnp.float32),
            pltpu.SemaphoreType.DMA,
        ],
        compiler_params=cp,
    )(stu_f, tea_f, unc_f)


def _tc_ent_kernel(stu_ref, tea_ref, unc_ref, sum_ref, cnt_ref):
    topo = 1.0 - jnp.abs(stu_ref[...] - tea_ref[...])
    conf = 1.0 - unc_ref[...]
    mask = (conf > 0.8).astype(jnp.float32)
    tc = jnp.clip(topo, EPS, 1.0 - EPS)
    ent = -(tc * jnp.log(tc + EPS))
    sum_ref[0, 0] = jnp.sum(ent * mask)
    cnt_ref[0, 0] = jnp.sum(mask)


def _tc_ent(stu, tea, unc):
    return pl.pallas_call(
        _tc_ent_kernel,
        out_shape=(jax.ShapeDtypeStruct((1, 1), jnp.float32),
                   jax.ShapeDtypeStruct((1, 1), jnp.float32)),
        out_specs=(pl.BlockSpec(memory_space=pltpu.SMEM),
                   pl.BlockSpec(memory_space=pltpu.SMEM)),
    )(stu, tea, unc)


def _tc_final_kernel(parts_ref, esum_ref, ecnt_ref, out_ref):
    hist = jnp.sum(parts_ref[...], axis=0)
    total = jnp.sum(hist)
    jp = hist / (total + EPS)
    mt = jnp.sum(jp, axis=1, keepdims=True)
    mc = jnp.sum(jp, axis=0, keepdims=True)
    outer = mt * mc
    contrib = jnp.where(jp > EPS, jp * jnp.log(jp / (outer + EPS) + EPS), 0.0)
    mi = jnp.sum(contrib)
    mi = jnp.where(total < EPS, jnp.float32(0.0), mi)
    esum = esum_ref[0, 0]
    cnt = ecnt_ref[0, 0]
    ent_loss = jnp.where(cnt > 10.0, esum / jnp.maximum(cnt, 1.0),
                         jnp.float32(0.0))
    out_ref[0, 0] = -mi + 0.1 * ent_loss


def _tc_final(parts, esum, ecnt):
    return pl.pallas_call(
        _tc_final_kernel,
        out_shape=jax.ShapeDtypeStruct((1, 1), jnp.float32),
        in_specs=(pl.BlockSpec(),
                  pl.BlockSpec(memory_space=pltpu.SMEM),
                  pl.BlockSpec(memory_space=pltpu.SMEM)),
        out_specs=pl.BlockSpec(memory_space=pltpu.SMEM),
    )(parts, esum, ecnt)


def kernel(stu_tensor, tea_tensor, stu_uncertainty):
    stu_f = stu_tensor.reshape(-1)
    tea_f = tea_tensor.reshape(-1)
    unc_f = stu_uncertainty.reshape(-1)
    parts = _sc_hist(stu_f, tea_f, unc_f)
    esum, ecnt = _tc_ent(stu_tensor, tea_tensor, stu_uncertainty)
    parts3 = parts.reshape(NWORK, NB, NB)
    out = _tc_final(parts3, esum, ecnt)
    return out.reshape(())


# speedup vs baseline: 20.9345x; 1.4566x over previous
import dataclasses

import jax
import jax.numpy as jnp
from jax import lax
from jax.experimental import pallas as pl
from jax.experimental.pallas import tpu as pltpu
from jax.experimental.pallas import tpu_sc as plsc

NB = 32
EPS = 1e-08
H = W = 512
N = H * W
NWORK = 32
SHARD = N // NWORK
LANES = 16
HIST = NB * NB


def _sc_hist_body(stu_hbm, tea_hbm, unc_hbm, out_hbm,
                  s_v, t_v, u_v, hist_v, sem):
    wid = lax.axis_index("s") * 2 + lax.axis_index("c")
    row0 = wid * (H // NWORK)

    rows = H // NWORK
    copies = [
        pltpu.async_copy(stu_hbm.at[pl.ds(row0, rows)], s_v, sem),
        pltpu.async_copy(tea_hbm.at[pl.ds(row0, rows)], t_v, sem),
        pltpu.async_copy(unc_hbm.at[pl.ds(row0, rows)], u_v, sem),
    ]

    zeros = jnp.zeros((LANES,), jnp.float32)

    @plsc.parallel_loop(0, NB, 1, unroll=4)
    def _(rr):
        for cc in range(0, NB, LANES):
            hist_v[rr, pl.ds(cc, LANES)] = zeros

    for c in copies:
        c.wait()

    ones = jnp.ones((LANES,), jnp.float32)
    nbf = float(NB)

    @plsc.parallel_loop(0, SHARD // LANES, 1, unroll=4)
    def _(m):
        r = m >> 5
        c = (m & 31) << 4
        s = s_v[r, pl.ds(c, LANES)]
        t = t_v[r, pl.ds(c, LANES)]
        u = u_v[r, pl.ds(c, LANES)]
        topo = 1.0 - jnp.abs(s - t)
        conf = 1.0 - u
        ti = (topo * nbf).astype(jnp.int32)
        cj = (conf * nbf).astype(jnp.int32)
        valid = (topo < 1.0) & (conf < 1.0)
        plsc.addupdate_scatter(hist_v, [ti, cj], ones, mask=valid)

    pltpu.sync_copy(hist_v, out_hbm.at[wid])


def _sc_hist(stu_f, tea_f, unc_f):
    mesh = plsc.VectorSubcoreMesh(core_axis_name="c", subcore_axis_name="s")
    cp = pltpu.CompilerParams()
    if "needs_layout_passes" in pltpu.CompilerParams.__dataclass_fields__:
        cp = dataclasses.replace(cp, needs_layout_passes=False)
    return pl.kernel(
        _sc_hist_body,
        out_type=jax.ShapeDtypeStruct((NWORK, NB, NB), jnp.float32),
        mesh=mesh,
        scratch_types=[
            pltpu.VMEM((H // NWORK, W), jnp.float32),
            pltpu.VMEM((H // NWORK, W), jnp.float32),
            pltpu.VMEM((H // NWORK, W), jnp.float32),
            pltpu.VMEM((NB, NB), jnp.float32),
            pltpu.SemaphoreType.DMA,
        ],
        compiler_params=cp,
    )(stu_f, tea_f, unc_f)


def _tc_ent_kernel(stu_ref, tea_ref, unc_ref, sum_ref, cnt_ref):
    topo = 1.0 - jnp.abs(stu_ref[...] - tea_ref[...])
    conf = 1.0 - unc_ref[...]
    mask = (conf > 0.8).astype(jnp.float32)
    tc = jnp.clip(topo, EPS, 1.0 - EPS)
    ent = -(tc * jnp.log(tc + EPS))
    sum_ref[0, 0] = jnp.sum(ent * mask)
    cnt_ref[0, 0] = jnp.sum(mask)


def _tc_ent(stu, tea, unc):
    return pl.pallas_call(
        _tc_ent_kernel,
        out_shape=(jax.ShapeDtypeStruct((1, 1), jnp.float32),
                   jax.ShapeDtypeStruct((1, 1), jnp.float32)),
        out_specs=(pl.BlockSpec(memory_space=pltpu.SMEM),
                   pl.BlockSpec(memory_space=pltpu.SMEM)),
    )(stu, tea, unc)


def _tc_final_kernel(parts_ref, esum_ref, ecnt_ref, out_ref):
    hist = jnp.sum(parts_ref[...], axis=0)
    total = jnp.sum(hist)
    jp = hist / (total + EPS)
    mt = jnp.sum(jp, axis=1, keepdims=True)
    mc = jnp.sum(jp, axis=0, keepdims=True)
    outer = mt * mc
    contrib = jnp.where(jp > EPS, jp * jnp.log(jp / (outer + EPS) + EPS), 0.0)
    mi = jnp.sum(contrib)
    mi = jnp.where(total < EPS, jnp.float32(0.0), mi)
    esum = esum_ref[0, 0]
    cnt = ecnt_ref[0, 0]
    ent_loss = jnp.where(cnt > 10.0, esum / jnp.maximum(cnt, 1.0),
                         jnp.float32(0.0))
    out_ref[0, 0] = -mi + 0.1 * ent_loss


def _tc_final(parts, esum, ecnt):
    return pl.pallas_call(
        _tc_final_kernel,
        out_shape=jax.ShapeDtypeStruct((1, 1), jnp.float32),
        in_specs=(pl.BlockSpec(),
                  pl.BlockSpec(memory_space=pltpu.SMEM),
                  pl.BlockSpec(memory_space=pltpu.SMEM)),
        out_specs=pl.BlockSpec(memory_space=pltpu.SMEM),
    )(parts, esum, ecnt)


def kernel(stu_tensor, tea_tensor, stu_uncertainty):
    parts = _sc_hist(stu_tensor, tea_tensor, stu_uncertainty)
    esum, ecnt = _tc_ent(stu_tensor, tea_tensor, stu_uncertainty)
    out = _tc_final(parts, esum, ecnt)
    return out.reshape(())
